# Initial kernel scaffold; baseline (speedup 1.0000x reference)
import jax
import jax.numpy as jnp
from jax.experimental import pallas as pl


def kernel(x_user, x_item, edge_index_u2i, edge_index_i2u, time_user, time_item, seed_time, batch_user, batch_item, W_enc_user, b_enc_user, W_enc_item, b_enc_item, W_time, b_time, W_root_u1, W_nbr_u1, W_root_i1, W_nbr_i1, W_root_u2, W_nbr_u2, W_root_i2, W_nbr_i2, gamma, beta, W_head, b_head):
    raise NotImplementedError("write your pallas kernel here")


# speedup vs baseline: 4.8439x; 4.8439x over previous
import jax
import jax.numpy as jnp
from jax import lax
from jax.experimental import pallas as pl
from jax.experimental.pallas import tpu as pltpu
from jax.experimental.pallas import tpu_sc as plsc

N = 10000
E = 320000
C = 128
B = 1024
OUT = 1
TDIM = 16
CP = C + 16
DUM = 16
NC, NS = 2, 16
NW = NC * NS
EPT = E // NW
K = 80
ITERS = EPT // K
RB = 1000
RN = N
RBD = B + DUM


def _seg_kernel(jobs):
    n = len(jobs)
    mesh = plsc.VectorSubcoreMesh(core_axis_name="c", subcore_axis_name="s")
    out_type = tuple(
        jax.ShapeDtypeStruct((NC, rows, CP), jnp.float32) for rows, _ in jobs
    )
    scratch = [
        pltpu.VMEM((ITERS, K), jnp.int32),
        pltpu.VMEM((ITERS, K), jnp.int32),
        pltpu.VMEM((K, CP), jnp.float32),
        pltpu.SemaphoreType.DMA,
    ] + [pltpu.VMEM_SHARED((rows, CP), jnp.float32) for rows, _ in jobs]

    def body(*refs):
        tables = refs[:n]
        srcs = refs[n:2 * n]
        dsts = refs[2 * n:3 * n]
        zeros = refs[3 * n]
        outs = refs[3 * n + 1:4 * n + 1]
        srcv, dstv, rowsv, sem = refs[4 * n + 1:4 * n + 5]
        accs = refs[4 * n + 5:]

        c = lax.axis_index("c")
        s = lax.axis_index("s")
        wid = c * NS + s

        for j, (rows, _) in enumerate(jobs):
            rpt = rows // NS
            sl = pl.ds(s * rpt, rpt)
            pltpu.sync_copy(zeros.at[sl], accs[j].at[sl])
        plsc.subcore_barrier()

        for j, (rows, clamp) in enumerate(jobs):
            pltpu.sync_copy(srcs[j].at[wid], srcv)
            pltpu.sync_copy(dsts[j].at[wid], dstv)
            if clamp:
                def cbody(i, carry):
                    for t in range(K // 16):
                        v = dstv[i, pl.ds(t * 16, 16)]
                        dstv[i, pl.ds(t * 16, 16)] = jnp.where(
                            v < B, v, B + (v & (DUM - 1)))
                    return carry
                lax.fori_loop(0, ITERS, cbody, 0)

            def ebody(i, carry, j=j):
                pltpu.async_copy(tables[j].at[srcv.at[i]], rowsv, sem).wait()
                pltpu.sync_copy(rowsv, accs[j].at[dstv.at[i]], add=True)
                return carry
            lax.fori_loop(0, ITERS, ebody, 0)
        plsc.subcore_barrier()

        for j, (rows, _) in enumerate(jobs):
            rpt = rows // NS
            sl = pl.ds(s * rpt, rpt)
            pltpu.sync_copy(accs[j].at[sl], outs[j].at[c].at[sl])

    return pl.kernel(
        body, out_type=out_type, mesh=mesh, scratch_types=scratch,
        compiler_params=pltpu.CompilerParams(use_tc_tiling_on_sc=False))


_sc1 = _seg_kernel(((RN, False), (RBD, True)))
_sc2 = _seg_kernel(((RBD, True),))


def _tc1_body(xu, xi, tu, ti, bu, bi, st, fr,
              Weu, beu, Wei, bei, Wt, bt, hu_o, hi_o):
    for x, t, b, We, be, out in ((xu, tu, bu, Weu, beu, hu_o),
                                 (xi, ti, bi, Wei, bei, hi_o)):
        eq = b[...] == lax.broadcasted_iota(jnp.int32, (RB, B), 1)
        rel = (jnp.sum(jnp.where(eq, st[...], 0.0), axis=1, keepdims=True)
               - t[...].astype(jnp.float32))
        pe = jnp.sin(rel * fr[...])
        out[...] = (jnp.maximum(x[...] @ We[...] + be[...], 0.0)
                    + pe @ Wt[...] + bt[...])


_tc1 = pl.pallas_call(
    _tc1_body,
    grid=(N // RB,),
    in_specs=[
        pl.BlockSpec((RB, C), lambda i: (i, 0)),
        pl.BlockSpec((RB, C), lambda i: (i, 0)),
        pl.BlockSpec((RB, 1), lambda i: (i, 0)),
        pl.BlockSpec((RB, 1), lambda i: (i, 0)),
        pl.BlockSpec((RB, 1), lambda i: (i, 0)),
        pl.BlockSpec((RB, 1), lambda i: (i, 0)),
        pl.BlockSpec((1, B), lambda i: (0, 0)),
        pl.BlockSpec((1, TDIM), lambda i: (0, 0)),
        pl.BlockSpec((C, CP), lambda i: (0, 0)),
        pl.BlockSpec((1, CP), lambda i: (0, 0)),
        pl.BlockSpec((C, CP), lambda i: (0, 0)),
        pl.BlockSpec((1, CP), lambda i: (0, 0)),
        pl.BlockSpec((TDIM, CP), lambda i: (0, 0)),
        pl.BlockSpec((1, CP), lambda i: (0, 0)),
    ],
    out_specs=[pl.BlockSpec((RB, CP), lambda i: (i, 0))] * 2,
    out_shape=[jax.ShapeDtypeStruct((N, CP), jnp.float32)] * 2,
)


def _tc2_body(hi, acci, Wr, Wn, colb, out):
    a = acci[...]
    ssum = a[0] + a[1]
    recip = 1.0 / jnp.maximum(ssum[:, C:C + 1], 1.0)
    mean = ssum * recip
    out[...] = jnp.maximum(hi[...] @ Wr[...] + mean @ Wn[...] + colb[...], 0.0)


_tc2 = pl.pallas_call(
    _tc2_body,
    grid=(N // RB,),
    in_specs=[
        pl.BlockSpec((RB, CP), lambda i: (i, 0)),
        pl.BlockSpec((NC, RB, CP), lambda i: (0, i, 0)),
        pl.BlockSpec((CP, CP), lambda i: (0, 0)),
        pl.BlockSpec((CP, CP), lambda i: (0, 0)),
        pl.BlockSpec((1, CP), lambda i: (0, 0)),
    ],
    out_specs=pl.BlockSpec((RB, CP), lambda i: (i, 0)),
    out_shape=jax.ShapeDtypeStruct((N, CP), jnp.float32),
)


def _tc3_body(hu, accu, accu2, Wr1, Wn1, Wr2, Wn2, gam, bet, Wh, bh, out):
    au = accu[...]
    a1 = au[0] + au[1]
    mu1 = a1 * (1.0 / jnp.maximum(a1[:, C:C + 1], 1.0))
    nh_u = jnp.maximum(hu[...] @ Wr1[...] + mu1 @ Wn1[...], 0.0)
    av = accu2[...]
    a2 = av[0] + av[1]
    mu2 = a2 * (1.0 / jnp.maximum(a2[:, C:C + 1], 1.0))
    ou = nh_u @ Wr2[...] + mu2 @ Wn2[...]
    m = jnp.mean(ou, axis=0, keepdims=True)
    v = jnp.mean((ou - m) ** 2, axis=0, keepdims=True)
    xn = (ou - m) / jnp.sqrt(v + 1e-5)
    out[...] = (xn * gam[...] + bet[...]) @ Wh[...] + bh[...]


_tc3 = pl.pallas_call(
    _tc3_body,
    grid=(1,),
    in_specs=[
        pl.BlockSpec((B, CP), lambda i: (0, 0)),
        pl.BlockSpec((NC, B, CP), lambda i: (0, 0, 0)),
        pl.BlockSpec((NC, B, CP), lambda i: (0, 0, 0)),
        pl.BlockSpec((CP, C), lambda i: (0, 0)),
        pl.BlockSpec((CP, C), lambda i: (0, 0)),
        pl.BlockSpec((C, C), lambda i: (0, 0)),
        pl.BlockSpec((CP, C), lambda i: (0, 0)),
        pl.BlockSpec((1, C), lambda i: (0, 0)),
        pl.BlockSpec((1, C), lambda i: (0, 0)),
        pl.BlockSpec((C, OUT), lambda i: (0, 0)),
        pl.BlockSpec((1, OUT), lambda i: (0, 0)),
    ],
    out_specs=pl.BlockSpec((B, OUT), lambda i: (0, 0)),
    out_shape=jax.ShapeDtypeStruct((B, OUT), jnp.float32),
)


def _pad_out(w):
    return jnp.zeros((w.shape[0], CP), jnp.float32).at[:, :C].set(w)


def _pad_rows(w):
    return jnp.zeros((CP, w.shape[1]), jnp.float32).at[:C, :].set(w)


def kernel(x_user, x_item, edge_index_u2i, edge_index_i2u, time_user,
           time_item, seed_time, batch_user, batch_item, W_enc_user,
           b_enc_user, W_enc_item, b_enc_item, W_time, b_time, W_root_u1,
           W_nbr_u1, W_root_i1, W_nbr_i1, W_root_u2, W_nbr_u2, W_root_i2,
           W_nbr_i2, gamma, beta, W_head, b_head):
    del W_root_i2, W_nbr_i2
    f32 = jnp.float32
    freqs = (1.0 / (10000.0 ** (jnp.arange(TDIM, dtype=f32) / TDIM)))
    ones_col = jnp.zeros((1, CP), f32).at[0, C].set(1.0)

    hu, hi = _tc1(
        x_user, x_item,
        time_user.reshape(N, 1), time_item.reshape(N, 1),
        batch_user.reshape(N, 1), batch_item.reshape(N, 1),
        seed_time.astype(f32).reshape(1, B), freqs.reshape(1, TDIM),
        _pad_out(W_enc_user), _pad_out(b_enc_user.reshape(1, C)) + ones_col,
        _pad_out(W_enc_item), _pad_out(b_enc_item.reshape(1, C)) + ones_col,
        _pad_out(W_time), _pad_out(b_time.reshape(1, C)),
    )

    su2i = edge_index_u2i[0].reshape(NW, ITERS, K)
    du2i = edge_index_u2i[1].reshape(NW, ITERS, K)
    si2u = edge_index_i2u[0].reshape(NW, ITERS, K)
    di2u = edge_index_i2u[1].reshape(NW, ITERS, K)
    zeros = jnp.zeros((RN, CP), f32)

    acc_i, acc_u = _sc1(hu, hi, su2i, si2u, du2i, di2u, zeros)

    nhi = _tc2(hi, acc_i, _pad_rows(_pad_out(W_root_i1)),
               _pad_rows(_pad_out(W_nbr_i1)), ones_col)

    acc_u2 = _sc2(nhi, si2u, di2u, zeros)
    if isinstance(acc_u2, (tuple, list)):
        (acc_u2,) = acc_u2

    return _tc3(hu, acc_u, acc_u2,
                _pad_rows(W_root_u1), _pad_rows(W_nbr_u1),
                W_root_u2, _pad_rows(W_nbr_u2),
                gamma.reshape(1, C), beta.reshape(1, C),
                W_head, b_head.reshape(1, OUT))
